# baseline (speedup 1.0000x reference)
import functools

import jax
import jax.numpy as jnp
from jax import lax
from jax.experimental import pallas as pl
from jax.experimental.pallas import tpu as pltpu
from jax.experimental.pallas import tpu_sc as plsc

NU = 10000
NI = 10000
NE = 320000
NT = 5
HID = 128
OUTD = 64
NC = 2
NS = 16
L = 16
TAB = NU * NT
TABP = 50176
CH = 80
EPT32 = NE // (NC * NS)
EPT16 = NE // NS
STRIPE = TABP // NS
SUP = 800

_sds = jax.ShapeDtypeStruct


def _mesh():
    return plsc.VectorSubcoreMesh(
        core_axis_name="c", subcore_axis_name="s",
        num_cores=NC, num_subcores=NS)


@functools.cache
def _deg_kernel():
  return functools.partial(
    pl.kernel,
    out_type=(_sds((NE,), jnp.int32), _sds((NE,), jnp.int32),
              _sds((NC * TABP,), jnp.float32), _sds((NC * TABP,), jnp.float32)),
    mesh=_mesh(),
    scratch_types=[
        pltpu.VMEM((EPT32,), jnp.int32),
        pltpu.VMEM((EPT32,), jnp.int32),
        pltpu.VMEM((EPT32,), jnp.int32),
        pltpu.VMEM((EPT32,), jnp.int32),
        pltpu.VMEM((EPT32,), jnp.int32),
        pltpu.VMEM((CH,), jnp.int32),
        pltpu.VMEM((CH,), jnp.int32),
        pltpu.VMEM((CH,), jnp.float32),
        pltpu.VMEM((STRIPE,), jnp.float32),
        pltpu.VMEM_SHARED((TABP,), jnp.float32),
        pltpu.VMEM_SHARED((TABP,), jnp.float32),
    ],
  )(_deg_body)


def _deg_body(u_hbm, i_hbm, t_hbm, z_hbm, ku_hbm, ki_hbm, degu_hbm, degi_hbm,
              uu, ii, tt, kuf, kif, ku80, ki80, ones80, dbuf,
              degu_s, degi_s):
    cid = lax.axis_index("c")
    sid = lax.axis_index("s")
    wid = cid * NS + sid
    base = wid * EPT32

    one16 = jnp.ones((L,), jnp.float32)
    for g in range(CH // L):
        ones80[pl.ds(g * L, L)] = one16
    pltpu.sync_copy(z_hbm.at[pl.ds(0, STRIPE)], dbuf)
    pltpu.sync_copy(dbuf, degu_s.at[pl.ds(sid * STRIPE, STRIPE)])
    pltpu.sync_copy(dbuf, degi_s.at[pl.ds(sid * STRIPE, STRIPE)])
    plsc.subcore_barrier()

    pltpu.sync_copy(u_hbm.at[pl.ds(base, EPT32)], uu)
    pltpu.sync_copy(i_hbm.at[pl.ds(base, EPT32)], ii)
    pltpu.sync_copy(t_hbm.at[pl.ds(base, EPT32)], tt)

    def chunk(c, _):
        for g in range(CH // L):
            o = c * CH + g * L
            uv = uu[pl.ds(o, L)]
            iv = ii[pl.ds(o, L)]
            tv = tt[pl.ds(o, L)]
            kuv = uv * NT + tv
            kiv = iv * NT + tv
            kuf[pl.ds(o, L)] = kuv
            kif[pl.ds(o, L)] = kiv
            ku80[pl.ds(g * L, L)] = kuv
            ki80[pl.ds(g * L, L)] = kiv
        pltpu.sync_copy(ones80, degu_s.at[ku80], add=True)
        pltpu.sync_copy(ones80, degi_s.at[ki80], add=True)
        return _

    lax.fori_loop(0, EPT32 // CH, chunk, None)
    pltpu.sync_copy(kuf, ku_hbm.at[pl.ds(base, EPT32)])
    pltpu.sync_copy(kif, ki_hbm.at[pl.ds(base, EPT32)])
    plsc.subcore_barrier()

    pltpu.sync_copy(degu_s.at[pl.ds(sid * STRIPE, STRIPE)], dbuf)
    pltpu.sync_copy(dbuf,
                    degu_hbm.at[pl.ds(cid * TABP + sid * STRIPE, STRIPE)])
    pltpu.sync_copy(degi_s.at[pl.ds(sid * STRIPE, STRIPE)], dbuf)
    pltpu.sync_copy(dbuf,
                    degi_hbm.at[pl.ds(cid * TABP + sid * STRIPE, STRIPE)])


def _coef_body(du_ref, di_ref, cu_ref, ci_ref):
    du = du_ref[0] + du_ref[1]
    di = di_ref[0] + di_ref[1]
    cu_ref[...] = lax.rsqrt(jnp.maximum(du, 1.0))
    ci_ref[...] = lax.rsqrt(jnp.maximum(di, 1.0))


def _coef_call(degu, degi):
    r = TABP // HID
    return pl.pallas_call(
        _coef_body,
        out_shape=(_sds((r, HID), jnp.float32), _sds((r, HID), jnp.float32)),
    )(degu.reshape(NC, r, HID), degi.reshape(NC, r, HID))


@functools.cache
def _ce_kernel():
  return functools.partial(
    pl.kernel,
    out_type=_sds((NE,), jnp.float32),
    mesh=_mesh(),
    scratch_types=[
        pltpu.VMEM((EPT32,), jnp.int32),
        pltpu.VMEM((EPT32,), jnp.int32),
        pltpu.VMEM((EPT32,), jnp.float32),
        pltpu.VMEM((CH,), jnp.float32),
        pltpu.VMEM((CH,), jnp.float32),
        pltpu.SemaphoreType.DMA,
    ],
  )(_ce_body)


def _ce_body(ku_hbm, ki_hbm, cu_hbm, ci_hbm, ce_hbm,
             kuf, kif, cef, gu, gi, sem):
    cid = lax.axis_index("c")
    sid = lax.axis_index("s")
    base = (cid * NS + sid) * EPT32
    pltpu.sync_copy(ku_hbm.at[pl.ds(base, EPT32)], kuf)
    pltpu.sync_copy(ki_hbm.at[pl.ds(base, EPT32)], kif)

    def chunk(c, _):
        pltpu.async_copy(cu_hbm.at[kuf.at[pl.ds(c * CH, CH)]], gu, sem).wait()
        pltpu.async_copy(ci_hbm.at[kif.at[pl.ds(c * CH, CH)]], gi, sem).wait()
        for g in range(CH // L):
            cef[pl.ds(c * CH + g * L, L)] = (
                gu[pl.ds(g * L, L)] * gi[pl.ds(g * L, L)])
        return _

    lax.fori_loop(0, EPT32 // CH, chunk, None)
    pltpu.sync_copy(cef, ce_hbm.at[pl.ds(base, EPT32)])


def _bcast_lane(v16, j):
    idx = jnp.full((L, 1), j, jnp.int32)
    dnums = lax.GatherDimensionNumbers(
        offset_dims=(), collapsed_slice_dims=(0,), start_index_map=(0,))
    return lax.gather(v16, idx, dnums, (1,),
                      mode=lax.GatherScatterMode.PROMISE_IN_BOUNDS)


@functools.cache
def _msg_kernel():
  return functools.partial(
    pl.kernel,
    out_type=_sds((NU + NI, HID), jnp.float32),
    mesh=_mesh(),
    scratch_types=[
        pltpu.VMEM((SUP,), jnp.int32),
        pltpu.VMEM((SUP,), jnp.int32),
        pltpu.VMEM((SUP,), jnp.float32),
        pltpu.VMEM((CH,), jnp.int32),
        pltpu.VMEM((CH,), jnp.int32),
        pltpu.VMEM((CH, HID), jnp.float32),
        pltpu.VMEM((CH, HID), jnp.float32),
        pltpu.VMEM((CH, HID), jnp.float32),
        pltpu.VMEM((CH, HID), jnp.float32),
        pltpu.VMEM_SHARED((NU, HID), jnp.float32),
        pltpu.SemaphoreType.DMA,
        pltpu.SemaphoreType.DMA,
        pltpu.SemaphoreType.DMA,
        pltpu.SemaphoreType.DMA,
    ],
  )(_msg_body)


def _msg_body(hi_hbm, hu_hbm, ki_hbm, ku_hbm, ui_hbm, ii_hbm, ce_hbm, z_hbm,
              out_hbm, gidx, sidx, cef, s80a, s80b, ga, gb, sa, sb, agg_s,
              gsa, gsb, ssa, ssb):
    cid = lax.axis_index("c")
    sid = lax.axis_index("s")
    base = sid * EPT16
    r0 = sid * 640
    nch = jnp.where(sid == NS - 1, 5, 8)
    npair = SUP // (2 * CH)

    def run(tab_hbm, g_hbm, s_hbm, out_base):
        pltpu.sync_copy(z_hbm, ga)

        def zc(k, _):
            pltpu.sync_copy(ga, agg_s.at[pl.ds(r0 + k * CH, CH)])
            return _

        lax.fori_loop(0, nch, zc, None)
        plsc.subcore_barrier()

        def scale(src, dst, sbuf, cbase):
            def grp(g, _):
                cev = cef[pl.ds(cbase + g * L, L)]
                sbuf[pl.ds(g * L, L)] = sidx[pl.ds(cbase + g * L, L)]
                for j in range(L):
                    sc = _bcast_lane(cev, j)
                    r = g * L + j
                    for k in range(HID // L):
                        dst[r, pl.ds(k * L, L)] = src[r, pl.ds(k * L, L)] * sc
                return _

            lax.fori_loop(0, CH // L, grp, None)

        def sup(s, _):
            b2 = base + s * SUP
            pltpu.sync_copy(g_hbm.at[pl.ds(b2, SUP)], gidx)
            pltpu.sync_copy(s_hbm.at[pl.ds(b2, SUP)], sidx)
            pltpu.sync_copy(ce_hbm.at[pl.ds(b2, SUP)], cef)
            pltpu.async_copy(tab_hbm.at[gidx.at[pl.ds(0, CH)]], ga, gsa)
            pltpu.async_copy(tab_hbm.at[gidx.at[pl.ds(CH, CH)]], gb, gsb)

            def pair(p, _):
                c0 = 2 * p * CH
                c1 = c0 + CH
                pltpu.make_async_copy(
                    tab_hbm.at[gidx.at[pl.ds(c0, CH)]], ga, gsa).wait()

                @pl.when(p > 0)
                def _():
                    pltpu.make_async_copy(sa, agg_s.at[s80a], ssa).wait()

                scale(ga, sa, s80a, c0)
                pltpu.async_copy(sa, agg_s.at[s80a], ssa, add=True)

                @pl.when(p < npair - 1)
                def _():
                    pltpu.async_copy(
                        tab_hbm.at[gidx.at[pl.ds(c0 + 2 * CH, CH)]], ga, gsa)

                pltpu.make_async_copy(
                    tab_hbm.at[gidx.at[pl.ds(c1, CH)]], gb, gsb).wait()

                @pl.when(p > 0)
                def _():
                    pltpu.make_async_copy(sb, agg_s.at[s80b], ssb).wait()

                scale(gb, sb, s80b, c1)
                pltpu.async_copy(sb, agg_s.at[s80b], ssb, add=True)

                @pl.when(p < npair - 1)
                def _():
                    pltpu.async_copy(
                        tab_hbm.at[gidx.at[pl.ds(c1 + 2 * CH, CH)]], gb, gsb)

                return _

            lax.fori_loop(0, npair, pair, None)
            pltpu.make_async_copy(sa, agg_s.at[s80a], ssa).wait()
            pltpu.make_async_copy(sb, agg_s.at[s80b], ssb).wait()
            return _

        lax.fori_loop(0, EPT16 // SUP, sup, None)
        plsc.subcore_barrier()

        def oc(k, _):
            pltpu.sync_copy(agg_s.at[pl.ds(r0 + k * CH, CH)], ga)
            pltpu.sync_copy(
                ga, out_hbm.at[pl.ds(out_base + r0 + k * CH, CH)])
            return _

        lax.fori_loop(0, nch, oc, None)
        plsc.subcore_barrier()

    @pl.when(cid == 0)
    def _():
        run(hi_hbm, ki_hbm, ui_hbm, 0)

    @pl.when(cid == 1)
    def _():
        run(hu_hbm, ku_hbm, ii_hbm, NU)


def _mm_call(x, wstack, bias, relu_flag):
    rtot = x.shape[0]
    d = wstack.shape[-1]
    br = 2000
    nb = rtot // br
    half = nb // 2

    def body(f_ref, x_ref, w_ref, b_ref, o_ref):
        xv = x_ref[...]
        xv = jnp.where(f_ref[0, 0] > 0.5, jnp.maximum(xv, 0.0), xv)
        acc = jnp.dot(xv, w_ref[0], preferred_element_type=jnp.float32,
                      precision=lax.Precision.HIGHEST)
        o_ref[...] = acc + b_ref[...]

    return pl.pallas_call(
        body,
        grid=(nb,),
        in_specs=[
            pl.BlockSpec((1, 1), lambda i: (0, 0)),
            pl.BlockSpec((br, HID), lambda i: (i, 0)),
            pl.BlockSpec((1, HID, d), lambda i: (i // half, 0, 0)),
            pl.BlockSpec((1, d), lambda i: (0, 0)),
        ],
        out_specs=pl.BlockSpec((br, d), lambda i: (i, 0)),
        out_shape=_sds((rtot, d), jnp.float32),
    )(relu_flag, x, wstack, bias)


def kernel(ufeats, ifeats, edge_index, edge_type, Wu, Wi, W_h, b_h):
    u_idx = edge_index[0].astype(jnp.int32)
    i_idx = edge_index[1].astype(jnp.int32)
    et = edge_type.astype(jnp.int32)

    x0 = jnp.concatenate([ufeats, ifeats], axis=0)
    wst = [jnp.stack([Wu[l].transpose(1, 0, 2).reshape(HID, NT * HID),
                      Wi[l].transpose(1, 0, 2).reshape(HID, NT * HID)])
           for l in range(2)]
    zb640 = jnp.zeros((1, NT * HID), jnp.float32)
    wout = jnp.stack([W_h, W_h])
    bout = b_h.reshape(1, OUTD)

    zflat = jnp.zeros((STRIPE,), jnp.float32)
    z2d = jnp.zeros((CH, HID), jnp.float32)

    ku, ki, degu, degi = _deg_kernel()(u_idx, i_idx, et, zflat)
    cu, ci = _coef_call(degu, degi)
    ce = _ce_kernel()(ku, ki, cu.reshape(TABP), ci.reshape(TABP))

    wsc = jnp.stack(wst)
    flags = jnp.array([0.0, 1.0]).reshape(2, 1, 1)

    def layer(x, per):
        w_l, f_l = per
        y = _mm_call(x, w_l, zb640, f_l)
        hu_tab = y[:NU].reshape(TAB, HID)
        hi_tab = y[NU:].reshape(TAB, HID)
        agg = _msg_kernel()(hi_tab, hu_tab, ki, ku, u_idx, i_idx, ce, z2d)
        return agg, None

    aggf, _ = lax.scan(layer, x0, (wsc, flags))
    outc = _mm_call(aggf, wout, bout, jnp.ones((1, 1), jnp.float32))
    return outc[:NU], outc[NU:]


# speedup vs baseline: 1.0849x; 1.0849x over previous
import functools

import jax
import jax.numpy as jnp
from jax import lax
from jax.experimental import pallas as pl
from jax.experimental.pallas import tpu as pltpu
from jax.experimental.pallas import tpu_sc as plsc

NU = 10000
NI = 10000
NE = 320000
NT = 5
HID = 128
OUTD = 64
NC = 2
NS = 16
L = 16
TAB = NU * NT
TABP = 50176
CH = 80
EPT32 = NE // (NC * NS)
EPT16 = NE // NS
STRIPE = TABP // NS
SUP = 800

_sds = jax.ShapeDtypeStruct


def _mesh():
    return plsc.VectorSubcoreMesh(
        core_axis_name="c", subcore_axis_name="s",
        num_cores=NC, num_subcores=NS)


@functools.cache
def _deg_kernel():
  return functools.partial(
    pl.kernel,
    out_type=(_sds((NE,), jnp.int32), _sds((NE,), jnp.int32),
              _sds((NC * TABP,), jnp.float32), _sds((NC * TABP,), jnp.float32)),
    mesh=_mesh(),
    scratch_types=[
        pltpu.VMEM((EPT32,), jnp.int32),
        pltpu.VMEM((EPT32,), jnp.int32),
        pltpu.VMEM((EPT32,), jnp.int32),
        pltpu.VMEM((EPT32,), jnp.int32),
        pltpu.VMEM((EPT32,), jnp.int32),
        pltpu.VMEM((CH,), jnp.int32),
        pltpu.VMEM((CH,), jnp.int32),
        pltpu.VMEM((CH,), jnp.int32),
        pltpu.VMEM((CH,), jnp.int32),
        pltpu.VMEM((CH,), jnp.float32),
        pltpu.VMEM((STRIPE,), jnp.float32),
        pltpu.VMEM_SHARED((TABP,), jnp.float32),
        pltpu.VMEM_SHARED((TABP,), jnp.float32),
        pltpu.SemaphoreType.DMA,
        pltpu.SemaphoreType.DMA,
    ],
  )(_deg_body)


def _deg_body(u_hbm, i_hbm, t_hbm, z_hbm, ku_hbm, ki_hbm, degu_hbm, degi_hbm,
              uu, ii, tt, kuf, kif, ku80a, ki80a, ku80b, ki80b, ones80, dbuf,
              degu_s, degi_s, sem_a, sem_b):
    cid = lax.axis_index("c")
    sid = lax.axis_index("s")
    wid = cid * NS + sid
    base = wid * EPT32

    one16 = jnp.ones((L,), jnp.float32)
    for g in range(CH // L):
        ones80[pl.ds(g * L, L)] = one16
    pltpu.sync_copy(z_hbm.at[pl.ds(0, STRIPE)], dbuf)
    pltpu.sync_copy(dbuf, degu_s.at[pl.ds(sid * STRIPE, STRIPE)])
    pltpu.sync_copy(dbuf, degi_s.at[pl.ds(sid * STRIPE, STRIPE)])
    plsc.subcore_barrier()

    pltpu.sync_copy(u_hbm.at[pl.ds(base, EPT32)], uu)
    pltpu.sync_copy(i_hbm.at[pl.ds(base, EPT32)], ii)
    pltpu.sync_copy(t_hbm.at[pl.ds(base, EPT32)], tt)

    def compute(c, kub, kib):
        for g in range(CH // L):
            o = c * CH + g * L
            uv = uu[pl.ds(o, L)]
            iv = ii[pl.ds(o, L)]
            tv = tt[pl.ds(o, L)]
            kuv = uv * NT + tv
            kiv = iv * NT + tv
            kuf[pl.ds(o, L)] = kuv
            kif[pl.ds(o, L)] = kiv
            kub[pl.ds(g * L, L)] = kuv
            kib[pl.ds(g * L, L)] = kiv

    def drain(kub, kib, sem):
        pltpu.make_async_copy(ones80, degu_s.at[kub], sem).wait()
        pltpu.make_async_copy(ones80, degi_s.at[kib], sem).wait()

    def fire(kub, kib, sem):
        pltpu.async_copy(ones80, degu_s.at[kub], sem, add=True)
        pltpu.async_copy(ones80, degi_s.at[kib], sem, add=True)

    def pair(q, _):
        @pl.when(q > 0)
        def _():
            drain(ku80a, ki80a, sem_a)

        compute(2 * q, ku80a, ki80a)
        fire(ku80a, ki80a, sem_a)

        @pl.when(q > 0)
        def _():
            drain(ku80b, ki80b, sem_b)

        compute(2 * q + 1, ku80b, ki80b)
        fire(ku80b, ki80b, sem_b)
        return _

    nfull = EPT32 // CH
    lax.fori_loop(0, nfull // 2, pair, None)
    drain(ku80a, ki80a, sem_a)
    compute(nfull - 1, ku80a, ki80a)
    fire(ku80a, ki80a, sem_a)
    drain(ku80a, ki80a, sem_a)
    drain(ku80b, ki80b, sem_b)
    pltpu.sync_copy(kuf, ku_hbm.at[pl.ds(base, EPT32)])
    pltpu.sync_copy(kif, ki_hbm.at[pl.ds(base, EPT32)])
    plsc.subcore_barrier()

    pltpu.sync_copy(degu_s.at[pl.ds(sid * STRIPE, STRIPE)], dbuf)
    pltpu.sync_copy(dbuf,
                    degu_hbm.at[pl.ds(cid * TABP + sid * STRIPE, STRIPE)])
    pltpu.sync_copy(degi_s.at[pl.ds(sid * STRIPE, STRIPE)], dbuf)
    pltpu.sync_copy(dbuf,
                    degi_hbm.at[pl.ds(cid * TABP + sid * STRIPE, STRIPE)])


def _coef_body(du_ref, di_ref, cu_ref, ci_ref):
    du = du_ref[0] + du_ref[1]
    di = di_ref[0] + di_ref[1]
    cu_ref[...] = lax.rsqrt(jnp.maximum(du, 1.0))
    ci_ref[...] = lax.rsqrt(jnp.maximum(di, 1.0))


def _coef_call(degu, degi):
    r = TABP // HID
    return pl.pallas_call(
        _coef_body,
        out_shape=(_sds((r, HID), jnp.float32), _sds((r, HID), jnp.float32)),
    )(degu.reshape(NC, r, HID), degi.reshape(NC, r, HID))


@functools.cache
def _ce_kernel():
  return functools.partial(
    pl.kernel,
    out_type=_sds((NE,), jnp.float32),
    mesh=_mesh(),
    scratch_types=[
        pltpu.VMEM((EPT32,), jnp.int32),
        pltpu.VMEM((EPT32,), jnp.int32),
        pltpu.VMEM((EPT32,), jnp.float32),
        pltpu.VMEM((CH,), jnp.float32),
        pltpu.VMEM((CH,), jnp.float32),
        pltpu.VMEM((CH,), jnp.float32),
        pltpu.VMEM((CH,), jnp.float32),
        pltpu.SemaphoreType.DMA,
        pltpu.SemaphoreType.DMA,
    ],
  )(_ce_body)


def _ce_body(ku_hbm, ki_hbm, cu_hbm, ci_hbm, ce_hbm,
             kuf, kif, cef, gua, gia, gub, gib, sem_a, sem_b):
    cid = lax.axis_index("c")
    sid = lax.axis_index("s")
    base = (cid * NS + sid) * EPT32
    pltpu.sync_copy(ku_hbm.at[pl.ds(base, EPT32)], kuf)
    pltpu.sync_copy(ki_hbm.at[pl.ds(base, EPT32)], kif)

    def fire(c, gu, gi, sem):
        pltpu.async_copy(cu_hbm.at[kuf.at[pl.ds(c * CH, CH)]], gu, sem)
        pltpu.async_copy(ci_hbm.at[kif.at[pl.ds(c * CH, CH)]], gi, sem)

    def drain(gu, gi, sem):
        pltpu.make_async_copy(cu_hbm.at[pl.ds(0, CH)], gu, sem).wait()
        pltpu.make_async_copy(ci_hbm.at[pl.ds(0, CH)], gi, sem).wait()

    def mult(c, gu, gi):
        for g in range(CH // L):
            cef[pl.ds(c * CH + g * L, L)] = (
                gu[pl.ds(g * L, L)] * gi[pl.ds(g * L, L)])

    fire(0, gua, gia, sem_a)

    def pair(q, _):
        c0 = 2 * q
        drain(gua, gia, sem_a)
        fire(c0 + 1, gub, gib, sem_b)
        mult(c0, gua, gia)
        drain(gub, gib, sem_b)
        fire(c0 + 2, gua, gia, sem_a)
        mult(c0 + 1, gub, gib)
        return _

    nfull = EPT32 // CH
    lax.fori_loop(0, nfull // 2, pair, None)
    drain(gua, gia, sem_a)
    mult(nfull - 1, gua, gia)
    pltpu.sync_copy(cef, ce_hbm.at[pl.ds(base, EPT32)])


def _bcast_lane(v16, j):
    idx = jnp.full((L, 1), j, jnp.int32)
    dnums = lax.GatherDimensionNumbers(
        offset_dims=(), collapsed_slice_dims=(0,), start_index_map=(0,))
    return lax.gather(v16, idx, dnums, (1,),
                      mode=lax.GatherScatterMode.PROMISE_IN_BOUNDS)


@functools.cache
def _msg_kernel():
  return functools.partial(
    pl.kernel,
    out_type=_sds((NU + NI, HID), jnp.float32),
    mesh=_mesh(),
    scratch_types=[
        pltpu.VMEM((SUP,), jnp.int32),
        pltpu.VMEM((SUP,), jnp.int32),
        pltpu.VMEM((SUP,), jnp.float32),
        pltpu.VMEM((CH,), jnp.int32),
        pltpu.VMEM((CH,), jnp.int32),
        pltpu.VMEM((CH, HID), jnp.float32),
        pltpu.VMEM((CH, HID), jnp.float32),
        pltpu.VMEM((CH, HID), jnp.float32),
        pltpu.VMEM((CH, HID), jnp.float32),
        pltpu.VMEM_SHARED((NU, HID), jnp.float32),
        pltpu.SemaphoreType.DMA,
        pltpu.SemaphoreType.DMA,
        pltpu.SemaphoreType.DMA,
        pltpu.SemaphoreType.DMA,
    ],
  )(_msg_body)


def _msg_body(hi_hbm, hu_hbm, ki_hbm, ku_hbm, ui_hbm, ii_hbm, ce_hbm, z_hbm,
              out_hbm, gidx, sidx, cef, s80a, s80b, ga, gb, sa, sb, agg_s,
              gsa, gsb, ssa, ssb):
    cid = lax.axis_index("c")
    sid = lax.axis_index("s")
    base = sid * EPT16
    r0 = sid * 640
    nch = jnp.where(sid == NS - 1, 5, 8)
    npair = SUP // (2 * CH)

    def run(tab_hbm, g_hbm, s_hbm, out_base):
        pltpu.sync_copy(z_hbm, ga)

        def zc(k, _):
            pltpu.sync_copy(ga, agg_s.at[pl.ds(r0 + k * CH, CH)])
            return _

        lax.fori_loop(0, nch, zc, None)
        plsc.subcore_barrier()

        def scale(src, dst, sbuf, cbase):
            def grp(g, _):
                cev = cef[pl.ds(cbase + g * L, L)]
                sbuf[pl.ds(g * L, L)] = sidx[pl.ds(cbase + g * L, L)]
                for j in range(L):
                    sc = _bcast_lane(cev, j)
                    r = g * L + j
                    for k in range(HID // L):
                        dst[r, pl.ds(k * L, L)] = src[r, pl.ds(k * L, L)] * sc
                return _

            lax.fori_loop(0, CH // L, grp, None)

        def sup(s, _):
            b2 = base + s * SUP
            pltpu.sync_copy(g_hbm.at[pl.ds(b2, SUP)], gidx)
            pltpu.sync_copy(s_hbm.at[pl.ds(b2, SUP)], sidx)
            pltpu.sync_copy(ce_hbm.at[pl.ds(b2, SUP)], cef)
            pltpu.async_copy(tab_hbm.at[gidx.at[pl.ds(0, CH)]], ga, gsa)
            pltpu.async_copy(tab_hbm.at[gidx.at[pl.ds(CH, CH)]], gb, gsb)

            def pair(p, _):
                c0 = 2 * p * CH
                c1 = c0 + CH
                pltpu.make_async_copy(
                    tab_hbm.at[gidx.at[pl.ds(c0, CH)]], ga, gsa).wait()

                @pl.when(p > 0)
                def _():
                    pltpu.make_async_copy(sa, agg_s.at[s80a], ssa).wait()

                scale(ga, sa, s80a, c0)
                pltpu.async_copy(sa, agg_s.at[s80a], ssa, add=True)

                @pl.when(p < npair - 1)
                def _():
                    pltpu.async_copy(
                        tab_hbm.at[gidx.at[pl.ds(c0 + 2 * CH, CH)]], ga, gsa)

                pltpu.make_async_copy(
                    tab_hbm.at[gidx.at[pl.ds(c1, CH)]], gb, gsb).wait()

                @pl.when(p > 0)
                def _():
                    pltpu.make_async_copy(sb, agg_s.at[s80b], ssb).wait()

                scale(gb, sb, s80b, c1)
                pltpu.async_copy(sb, agg_s.at[s80b], ssb, add=True)

                @pl.when(p < npair - 1)
                def _():
                    pltpu.async_copy(
                        tab_hbm.at[gidx.at[pl.ds(c1 + 2 * CH, CH)]], gb, gsb)

                return _

            lax.fori_loop(0, npair, pair, None)
            pltpu.make_async_copy(sa, agg_s.at[s80a], ssa).wait()
            pltpu.make_async_copy(sb, agg_s.at[s80b], ssb).wait()
            return _

        lax.fori_loop(0, EPT16 // SUP, sup, None)
        plsc.subcore_barrier()

        def oc(k, _):
            pltpu.sync_copy(agg_s.at[pl.ds(r0 + k * CH, CH)], ga)
            pltpu.sync_copy(
                ga, out_hbm.at[pl.ds(out_base + r0 + k * CH, CH)])
            return _

        lax.fori_loop(0, nch, oc, None)
        plsc.subcore_barrier()

    @pl.when(cid == 0)
    def _():
        run(hi_hbm, ki_hbm, ui_hbm, 0)

    @pl.when(cid == 1)
    def _():
        run(hu_hbm, ku_hbm, ii_hbm, NU)


def _mm_call(x, wstack, bias, relu_flag):
    rtot = x.shape[0]
    d = wstack.shape[-1]
    br = 2000
    nb = rtot // br
    half = nb // 2

    def body(f_ref, x_ref, w_ref, b_ref, o_ref):
        xv = x_ref[...]
        xv = jnp.where(f_ref[0, 0] > 0.5, jnp.maximum(xv, 0.0), xv)
        acc = jnp.dot(xv, w_ref[0], preferred_element_type=jnp.float32,
                      precision=lax.Precision.HIGHEST)
        o_ref[...] = acc + b_ref[...]

    return pl.pallas_call(
        body,
        grid=(nb,),
        in_specs=[
            pl.BlockSpec((1, 1), lambda i: (0, 0)),
            pl.BlockSpec((br, HID), lambda i: (i, 0)),
            pl.BlockSpec((1, HID, d), lambda i: (i // half, 0, 0)),
            pl.BlockSpec((1, d), lambda i: (0, 0)),
        ],
        out_specs=pl.BlockSpec((br, d), lambda i: (i, 0)),
        out_shape=_sds((rtot, d), jnp.float32),
    )(relu_flag, x, wstack, bias)


def kernel(ufeats, ifeats, edge_index, edge_type, Wu, Wi, W_h, b_h):
    u_idx = edge_index[0].astype(jnp.int32)
    i_idx = edge_index[1].astype(jnp.int32)
    et = edge_type.astype(jnp.int32)

    x0 = jnp.concatenate([ufeats, ifeats], axis=0)
    wst = [jnp.stack([Wu[l].transpose(1, 0, 2).reshape(HID, NT * HID),
                      Wi[l].transpose(1, 0, 2).reshape(HID, NT * HID)])
           for l in range(2)]
    zb640 = jnp.zeros((1, NT * HID), jnp.float32)
    wout = jnp.stack([W_h, W_h])
    bout = b_h.reshape(1, OUTD)

    zflat = jnp.zeros((STRIPE,), jnp.float32)
    z2d = jnp.zeros((CH, HID), jnp.float32)

    ku, ki, degu, degi = _deg_kernel()(u_idx, i_idx, et, zflat)
    cu, ci = _coef_call(degu, degi)
    ce = _ce_kernel()(ku, ki, cu.reshape(TABP), ci.reshape(TABP))

    wsc = jnp.stack(wst)
    flags = jnp.array([0.0, 1.0]).reshape(2, 1, 1)

    def layer(x, per):
        w_l, f_l = per
        y = _mm_call(x, w_l, zb640, f_l)
        hu_tab = y[:NU].reshape(TAB, HID)
        hi_tab = y[NU:].reshape(TAB, HID)
        agg = _msg_kernel()(hi_tab, hu_tab, ki, ku, u_idx, i_idx, ce, z2d)
        return agg, None

    aggf, _ = lax.scan(layer, x0, (wsc, flags))
    outc = _mm_call(aggf, wout, bout, jnp.ones((1, 1), jnp.float32))
    return outc[:NU], outc[NU:]


# speedup vs baseline: 1.1232x; 1.0353x over previous
import functools

import jax
import jax.numpy as jnp
from jax import lax
from jax.experimental import pallas as pl
from jax.experimental.pallas import tpu as pltpu
from jax.experimental.pallas import tpu_sc as plsc

NU = 10000
NI = 10000
NE = 320000
NT = 5
HID = 128
OUTD = 64
NC = 2
NS = 16
L = 16
TAB = NU * NT
TABP = 50176
CH = 80
EPT32 = NE // (NC * NS)
EPT16 = NE // NS
STRIPE = TABP // NS
SUP = 4000

_sds = jax.ShapeDtypeStruct


def _mesh():
    return plsc.VectorSubcoreMesh(
        core_axis_name="c", subcore_axis_name="s",
        num_cores=NC, num_subcores=NS)


@functools.cache
def _deg_kernel():
  return functools.partial(
    pl.kernel,
    out_type=(_sds((NE,), jnp.int32), _sds((NE,), jnp.int32),
              _sds((NC * TABP,), jnp.float32), _sds((NC * TABP,), jnp.float32)),
    mesh=_mesh(),
    scratch_types=[
        pltpu.VMEM((EPT32,), jnp.int32),
        pltpu.VMEM((EPT32,), jnp.int32),
        pltpu.VMEM((EPT32,), jnp.int32),
        pltpu.VMEM((EPT32,), jnp.int32),
        pltpu.VMEM((EPT32,), jnp.int32),
        pltpu.VMEM((CH,), jnp.int32),
        pltpu.VMEM((CH,), jnp.int32),
        pltpu.VMEM((CH,), jnp.int32),
        pltpu.VMEM((CH,), jnp.int32),
        pltpu.VMEM((CH,), jnp.float32),
        pltpu.VMEM((STRIPE,), jnp.float32),
        pltpu.VMEM_SHARED((TABP,), jnp.float32),
        pltpu.VMEM_SHARED((TABP,), jnp.float32),
        pltpu.SemaphoreType.DMA,
        pltpu.SemaphoreType.DMA,
    ],
  )(_deg_body)


def _deg_body(u_hbm, i_hbm, t_hbm, z_hbm, ku_hbm, ki_hbm, degu_hbm, degi_hbm,
              uu, ii, tt, kuf, kif, ku80a, ki80a, ku80b, ki80b, ones80, dbuf,
              degu_s, degi_s, sem_a, sem_b):
    cid = lax.axis_index("c")
    sid = lax.axis_index("s")
    wid = cid * NS + sid
    base = wid * EPT32

    one16 = jnp.ones((L,), jnp.float32)
    for g in range(CH // L):
        ones80[pl.ds(g * L, L)] = one16
    pltpu.sync_copy(z_hbm.at[pl.ds(0, STRIPE)], dbuf)
    pltpu.sync_copy(dbuf, degu_s.at[pl.ds(sid * STRIPE, STRIPE)])
    pltpu.sync_copy(dbuf, degi_s.at[pl.ds(sid * STRIPE, STRIPE)])
    plsc.subcore_barrier()

    pltpu.sync_copy(u_hbm.at[pl.ds(base, EPT32)], uu)
    pltpu.sync_copy(i_hbm.at[pl.ds(base, EPT32)], ii)
    pltpu.sync_copy(t_hbm.at[pl.ds(base, EPT32)], tt)

    def compute(c, kub, kib):
        for g in range(CH // L):
            o = c * CH + g * L
            uv = uu[pl.ds(o, L)]
            iv = ii[pl.ds(o, L)]
            tv = tt[pl.ds(o, L)]
            kuv = uv * NT + tv
            kiv = iv * NT + tv
            kuf[pl.ds(o, L)] = kuv
            kif[pl.ds(o, L)] = kiv
            kub[pl.ds(g * L, L)] = kuv
            kib[pl.ds(g * L, L)] = kiv

    def drain(kub, kib, sem):
        pltpu.make_async_copy(ones80, degu_s.at[kub], sem).wait()
        pltpu.make_async_copy(ones80, degi_s.at[kib], sem).wait()

    def fire(kub, kib, sem):
        pltpu.async_copy(ones80, degu_s.at[kub], sem, add=True)
        pltpu.async_copy(ones80, degi_s.at[kib], sem, add=True)

    def pair(q, _):
        @pl.when(q > 0)
        def _():
            drain(ku80a, ki80a, sem_a)

        compute(2 * q, ku80a, ki80a)
        fire(ku80a, ki80a, sem_a)

        @pl.when(q > 0)
        def _():
            drain(ku80b, ki80b, sem_b)

        compute(2 * q + 1, ku80b, ki80b)
        fire(ku80b, ki80b, sem_b)
        return _

    nfull = EPT32 // CH
    lax.fori_loop(0, nfull // 2, pair, None)
    drain(ku80a, ki80a, sem_a)
    compute(nfull - 1, ku80a, ki80a)
    fire(ku80a, ki80a, sem_a)
    drain(ku80a, ki80a, sem_a)
    drain(ku80b, ki80b, sem_b)
    pltpu.sync_copy(kuf, ku_hbm.at[pl.ds(base, EPT32)])
    pltpu.sync_copy(kif, ki_hbm.at[pl.ds(base, EPT32)])
    plsc.subcore_barrier()

    pltpu.sync_copy(degu_s.at[pl.ds(sid * STRIPE, STRIPE)], dbuf)
    pltpu.sync_copy(dbuf,
                    degu_hbm.at[pl.ds(cid * TABP + sid * STRIPE, STRIPE)])
    pltpu.sync_copy(degi_s.at[pl.ds(sid * STRIPE, STRIPE)], dbuf)
    pltpu.sync_copy(dbuf,
                    degi_hbm.at[pl.ds(cid * TABP + sid * STRIPE, STRIPE)])


def _coef_body(du_ref, di_ref, cu_ref, ci_ref):
    du = du_ref[0] + du_ref[1]
    di = di_ref[0] + di_ref[1]
    cu_ref[...] = lax.rsqrt(jnp.maximum(du, 1.0))
    ci_ref[...] = lax.rsqrt(jnp.maximum(di, 1.0))


def _coef_call(degu, degi):
    r = TABP // HID
    return pl.pallas_call(
        _coef_body,
        out_shape=(_sds((r, HID), jnp.float32), _sds((r, HID), jnp.float32)),
    )(degu.reshape(NC, r, HID), degi.reshape(NC, r, HID))


@functools.cache
def _ce_kernel():
  return functools.partial(
    pl.kernel,
    out_type=_sds((NE,), jnp.float32),
    mesh=_mesh(),
    scratch_types=[
        pltpu.VMEM((EPT32,), jnp.int32),
        pltpu.VMEM((EPT32,), jnp.int32),
        pltpu.VMEM((EPT32,), jnp.float32),
        pltpu.VMEM((CH,), jnp.float32),
        pltpu.VMEM((CH,), jnp.float32),
        pltpu.VMEM((CH,), jnp.float32),
        pltpu.VMEM((CH,), jnp.float32),
        pltpu.SemaphoreType.DMA,
        pltpu.SemaphoreType.DMA,
    ],
  )(_ce_body)


def _ce_body(ku_hbm, ki_hbm, cu_hbm, ci_hbm, ce_hbm,
             kuf, kif, cef, gua, gia, gub, gib, sem_a, sem_b):
    cid = lax.axis_index("c")
    sid = lax.axis_index("s")
    base = (cid * NS + sid) * EPT32
    pltpu.sync_copy(ku_hbm.at[pl.ds(base, EPT32)], kuf)
    pltpu.sync_copy(ki_hbm.at[pl.ds(base, EPT32)], kif)

    def fire(c, gu, gi, sem):
        pltpu.async_copy(cu_hbm.at[kuf.at[pl.ds(c * CH, CH)]], gu, sem)
        pltpu.async_copy(ci_hbm.at[kif.at[pl.ds(c * CH, CH)]], gi, sem)

    def drain(gu, gi, sem):
        pltpu.make_async_copy(cu_hbm.at[pl.ds(0, CH)], gu, sem).wait()
        pltpu.make_async_copy(ci_hbm.at[pl.ds(0, CH)], gi, sem).wait()

    def mult(c, gu, gi):
        for g in range(CH // L):
            cef[pl.ds(c * CH + g * L, L)] = (
                gu[pl.ds(g * L, L)] * gi[pl.ds(g * L, L)])

    fire(0, gua, gia, sem_a)

    def pair(q, _):
        c0 = 2 * q
        drain(gua, gia, sem_a)
        fire(c0 + 1, gub, gib, sem_b)
        mult(c0, gua, gia)
        drain(gub, gib, sem_b)
        fire(c0 + 2, gua, gia, sem_a)
        mult(c0 + 1, gub, gib)
        return _

    nfull = EPT32 // CH
    lax.fori_loop(0, nfull // 2, pair, None)
    drain(gua, gia, sem_a)
    mult(nfull - 1, gua, gia)
    pltpu.sync_copy(cef, ce_hbm.at[pl.ds(base, EPT32)])


def _bcast_lane(v16, j):
    idx = jnp.full((L, 1), j, jnp.int32)
    dnums = lax.GatherDimensionNumbers(
        offset_dims=(), collapsed_slice_dims=(0,), start_index_map=(0,))
    return lax.gather(v16, idx, dnums, (1,),
                      mode=lax.GatherScatterMode.PROMISE_IN_BOUNDS)


@functools.cache
def _msg_kernel():
  return functools.partial(
    pl.kernel,
    out_type=_sds((NU + NI, HID), jnp.float32),
    mesh=_mesh(),
    scratch_types=[
        pltpu.VMEM((EPT16,), jnp.int32),
        pltpu.VMEM((SUP,), jnp.int32),
        pltpu.VMEM((SUP,), jnp.float32),
        pltpu.VMEM((CH,), jnp.int32),
        pltpu.VMEM((CH, HID), jnp.float32),
        pltpu.VMEM((CH, HID), jnp.float32),
        pltpu.VMEM_SHARED((NU, HID), jnp.float32),
        pltpu.SemaphoreType.DMA,
        pltpu.SemaphoreType.DMA,
    ],
  )(_msg_body)


def _msg_body(hi_hbm, hu_hbm, ki_hbm, ku_hbm, ui_hbm, ii_hbm, ce_hbm, z_hbm,
              out_hbm, gidx, sidx, cef, s80, ga, gb, agg_s, gsa, gsb):
    cid = lax.axis_index("c")
    sid = lax.axis_index("s")
    base = sid * EPT16
    r0 = sid * 640
    nch = jnp.where(sid == NS - 1, 5, 8)
    npair = EPT16 // (2 * CH)
    npps = SUP // (2 * CH)

    def run(tab_hbm, g_hbm, s_hbm, out_base):
        pltpu.sync_copy(z_hbm, ga)

        def zc(k, _):
            pltpu.sync_copy(ga, agg_s.at[pl.ds(r0 + k * CH, CH)])
            return _

        lax.fori_loop(0, nch, zc, None)
        plsc.subcore_barrier()
        pltpu.sync_copy(g_hbm.at[pl.ds(base, EPT16)], gidx)

        def scale_scatter(buf, coff):
            def grp(g, _):
                cev = cef[pl.ds(coff + g * L, L)]
                s80[pl.ds(g * L, L)] = sidx[pl.ds(coff + g * L, L)]
                for j in range(L):
                    sc = _bcast_lane(cev, j)
                    r = g * L + j
                    for k in range(HID // L):
                        buf[r, pl.ds(k * L, L)] = buf[r, pl.ds(k * L, L)] * sc
                return _

            lax.fori_loop(0, CH // L, grp, None)
            pltpu.sync_copy(buf, agg_s.at[s80], add=True)

        pltpu.async_copy(tab_hbm.at[gidx.at[pl.ds(0, CH)]], ga, gsa)

        def pair(p, _):
            e0 = 2 * p * CH
            off0 = lax.rem(e0, SUP)

            @pl.when(lax.rem(p, npps) == 0)
            def _():
                pltpu.sync_copy(s_hbm.at[pl.ds(base + e0, SUP)], sidx)
                pltpu.sync_copy(ce_hbm.at[pl.ds(base + e0, SUP)], cef)

            pltpu.make_async_copy(
                tab_hbm.at[gidx.at[pl.ds(e0, CH)]], ga, gsa).wait()
            pltpu.async_copy(
                tab_hbm.at[gidx.at[pl.ds(e0 + CH, CH)]], gb, gsb)
            scale_scatter(ga, off0)

            @pl.when(p < npair - 1)
            def _():
                pltpu.async_copy(
                    tab_hbm.at[gidx.at[pl.ds(e0 + 2 * CH, CH)]], ga, gsa)

            pltpu.make_async_copy(
                tab_hbm.at[gidx.at[pl.ds(e0 + CH, CH)]], gb, gsb).wait()
            scale_scatter(gb, off0 + CH)
            return _

        lax.fori_loop(0, npair, pair, None)
        plsc.subcore_barrier()

        def oc(k, _):
            pltpu.sync_copy(agg_s.at[pl.ds(r0 + k * CH, CH)], ga)
            pltpu.sync_copy(
                ga, out_hbm.at[pl.ds(out_base + r0 + k * CH, CH)])
            return _

        lax.fori_loop(0, nch, oc, None)
        plsc.subcore_barrier()

    @pl.when(cid == 0)
    def _():
        run(hi_hbm, ki_hbm, ui_hbm, 0)

    @pl.when(cid == 1)
    def _():
        run(hu_hbm, ku_hbm, ii_hbm, NU)


def _mm_call(x, wstack, bias, relu_flag):
    rtot = x.shape[0]
    d = wstack.shape[-1]
    br = 2000
    nb = rtot // br
    half = nb // 2

    def body(f_ref, x_ref, w_ref, b_ref, o_ref):
        xv = x_ref[...]
        xv = jnp.where(f_ref[0, 0] > 0.5, jnp.maximum(xv, 0.0), xv)
        acc = jnp.dot(xv, w_ref[0], preferred_element_type=jnp.float32,
                      precision=lax.Precision.HIGHEST)
        o_ref[...] = acc + b_ref[...]

    return pl.pallas_call(
        body,
        grid=(nb,),
        in_specs=[
            pl.BlockSpec((1, 1), lambda i: (0, 0)),
            pl.BlockSpec((br, HID), lambda i: (i, 0)),
            pl.BlockSpec((1, HID, d), lambda i: (i // half, 0, 0)),
            pl.BlockSpec((1, d), lambda i: (0, 0)),
        ],
        out_specs=pl.BlockSpec((br, d), lambda i: (i, 0)),
        out_shape=_sds((rtot, d), jnp.float32),
    )(relu_flag, x, wstack, bias)


def kernel(ufeats, ifeats, edge_index, edge_type, Wu, Wi, W_h, b_h):
    u_idx = edge_index[0].astype(jnp.int32)
    i_idx = edge_index[1].astype(jnp.int32)
    et = edge_type.astype(jnp.int32)

    x0 = jnp.concatenate([ufeats, ifeats], axis=0)
    wst = [jnp.stack([Wu[l].transpose(1, 0, 2).reshape(HID, NT * HID),
                      Wi[l].transpose(1, 0, 2).reshape(HID, NT * HID)])
           for l in range(2)]
    zb640 = jnp.zeros((1, NT * HID), jnp.float32)
    wout = jnp.stack([W_h, W_h])
    bout = b_h.reshape(1, OUTD)

    zflat = jnp.zeros((STRIPE,), jnp.float32)
    z2d = jnp.zeros((CH, HID), jnp.float32)

    ku, ki, degu, degi = _deg_kernel()(u_idx, i_idx, et, zflat)
    cu, ci = _coef_call(degu, degi)
    ce = _ce_kernel()(ku, ki, cu.reshape(TABP), ci.reshape(TABP))

    wsc = jnp.stack(wst)
    flags = jnp.array([0.0, 1.0]).reshape(2, 1, 1)

    def layer(x, per):
        w_l, f_l = per
        y = _mm_call(x, w_l, zb640, f_l)
        hu_tab = y[:NU].reshape(TAB, HID)
        hi_tab = y[NU:].reshape(TAB, HID)
        agg = _msg_kernel()(hi_tab, hu_tab, ki, ku, u_idx, i_idx, ce, z2d)
        return agg, None

    aggf, _ = lax.scan(layer, x0, (wsc, flags))
    outc = _mm_call(aggf, wout, bout, jnp.ones((1, 1), jnp.float32))
    return outc[:NU], outc[NU:]


# speedup vs baseline: 1.2103x; 1.0776x over previous
import functools

import jax
import jax.numpy as jnp
from jax import lax
from jax.experimental import pallas as pl
from jax.experimental.pallas import tpu as pltpu
from jax.experimental.pallas import tpu_sc as plsc

NU = 10000
NI = 10000
NE = 320000
NT = 5
HID = 128
OUTD = 64
NC = 2
NS = 16
L = 16
TAB = NU * NT
TABP = 50176
CH = 80
EPT32 = NE // (NC * NS)
EPT16 = NE // NS
STRIPE = TABP // NS
SUP = 4000

_sds = jax.ShapeDtypeStruct


def _mesh():
    return plsc.VectorSubcoreMesh(
        core_axis_name="c", subcore_axis_name="s",
        num_cores=NC, num_subcores=NS)


@functools.cache
def _deg_kernel():
  return functools.partial(
    pl.kernel,
    out_type=(_sds((NE,), jnp.int32), _sds((NE,), jnp.int32),
              _sds((NC * TABP,), jnp.float32), _sds((NC * TABP,), jnp.float32)),
    mesh=_mesh(),
    scratch_types=[
        pltpu.VMEM((EPT32,), jnp.int32),
        pltpu.VMEM((EPT32,), jnp.int32),
        pltpu.VMEM((EPT32,), jnp.int32),
        pltpu.VMEM((EPT32,), jnp.int32),
        pltpu.VMEM((EPT32,), jnp.int32),
        pltpu.VMEM((4, CH), jnp.int32),
        pltpu.VMEM((4, CH), jnp.int32),
        pltpu.VMEM((CH,), jnp.float32),
        pltpu.VMEM((STRIPE,), jnp.float32),
        pltpu.VMEM_SHARED((TABP,), jnp.float32),
        pltpu.VMEM_SHARED((TABP,), jnp.float32),
        pltpu.SemaphoreType.DMA,
        pltpu.SemaphoreType.DMA,
        pltpu.SemaphoreType.DMA,
        pltpu.SemaphoreType.DMA,
    ],
  )(_deg_body)


def _deg_body(u_hbm, i_hbm, t_hbm, z_hbm, ku_hbm, ki_hbm, degu_hbm, degi_hbm,
              uu, ii, tt, kuf, kif, ku4, ki4, ones80, dbuf,
              degu_s, degi_s, sem0, sem1, sem2, sem3):
    cid = lax.axis_index("c")
    sid = lax.axis_index("s")
    wid = cid * NS + sid
    base = wid * EPT32

    one16 = jnp.ones((L,), jnp.float32)
    for g in range(CH // L):
        ones80[pl.ds(g * L, L)] = one16
    pltpu.sync_copy(z_hbm.at[pl.ds(0, STRIPE)], dbuf)
    pltpu.sync_copy(dbuf, degu_s.at[pl.ds(sid * STRIPE, STRIPE)])
    pltpu.sync_copy(dbuf, degi_s.at[pl.ds(sid * STRIPE, STRIPE)])
    plsc.subcore_barrier()

    pltpu.sync_copy(u_hbm.at[pl.ds(base, EPT32)], uu)
    pltpu.sync_copy(i_hbm.at[pl.ds(base, EPT32)], ii)
    pltpu.sync_copy(t_hbm.at[pl.ds(base, EPT32)], tt)

    sems = (sem0, sem1, sem2, sem3)

    def compute(c, j):
        for g in range(CH // L):
            o = c * CH + g * L
            uv = uu[pl.ds(o, L)]
            iv = ii[pl.ds(o, L)]
            tv = tt[pl.ds(o, L)]
            kuv = uv * NT + tv
            kiv = iv * NT + tv
            kuf[pl.ds(o, L)] = kuv
            kif[pl.ds(o, L)] = kiv
            ku4[j, pl.ds(g * L, L)] = kuv
            ki4[j, pl.ds(g * L, L)] = kiv

    def drain(j):
        pltpu.make_async_copy(ones80, degu_s.at[ku4.at[j]], sems[j]).wait()
        pltpu.make_async_copy(ones80, degi_s.at[ki4.at[j]], sems[j]).wait()

    def fire(j):
        pltpu.async_copy(ones80, degu_s.at[ku4.at[j]], sems[j], add=True)
        pltpu.async_copy(ones80, degi_s.at[ki4.at[j]], sems[j], add=True)

    def quad(q, _):
        for j in range(4):
            @pl.when(q > 0)
            def _():
                drain(j)

            compute(4 * q + j, j)
            fire(j)
        return _

    lax.fori_loop(0, 31, quad, None)
    drain(0)
    compute(124, 0)
    fire(0)
    for j in range(4):
        drain(j)
    pltpu.sync_copy(kuf, ku_hbm.at[pl.ds(base, EPT32)])
    pltpu.sync_copy(kif, ki_hbm.at[pl.ds(base, EPT32)])
    plsc.subcore_barrier()

    pltpu.sync_copy(degu_s.at[pl.ds(sid * STRIPE, STRIPE)], dbuf)
    pltpu.sync_copy(dbuf,
                    degu_hbm.at[pl.ds(cid * TABP + sid * STRIPE, STRIPE)])
    pltpu.sync_copy(degi_s.at[pl.ds(sid * STRIPE, STRIPE)], dbuf)
    pltpu.sync_copy(dbuf,
                    degi_hbm.at[pl.ds(cid * TABP + sid * STRIPE, STRIPE)])


def _coef_body(du_ref, di_ref, cu_ref, ci_ref):
    du = du_ref[0] + du_ref[1]
    di = di_ref[0] + di_ref[1]
    cu_ref[...] = lax.rsqrt(jnp.maximum(du, 1.0))
    ci_ref[...] = lax.rsqrt(jnp.maximum(di, 1.0))


def _coef_call(degu, degi):
    r = TABP // HID
    return pl.pallas_call(
        _coef_body,
        out_shape=(_sds((r, HID), jnp.float32), _sds((r, HID), jnp.float32)),
    )(degu.reshape(NC, r, HID), degi.reshape(NC, r, HID))


@functools.cache
def _ce_kernel():
  return functools.partial(
    pl.kernel,
    out_type=_sds((NE,), jnp.float32),
    mesh=_mesh(),
    scratch_types=[
        pltpu.VMEM((EPT32,), jnp.int32),
        pltpu.VMEM((EPT32,), jnp.int32),
        pltpu.VMEM((EPT32,), jnp.float32),
        pltpu.VMEM((4, CH), jnp.float32),
        pltpu.VMEM((4, CH), jnp.float32),
        pltpu.SemaphoreType.DMA,
        pltpu.SemaphoreType.DMA,
        pltpu.SemaphoreType.DMA,
        pltpu.SemaphoreType.DMA,
    ],
  )(_ce_body)


def _ce_body(ku_hbm, ki_hbm, cu_hbm, ci_hbm, ce_hbm,
             kuf, kif, cef, gu4, gi4, sem0, sem1, sem2, sem3):
    cid = lax.axis_index("c")
    sid = lax.axis_index("s")
    base = (cid * NS + sid) * EPT32
    sems = (sem0, sem1, sem2, sem3)
    pltpu.sync_copy(ku_hbm.at[pl.ds(base, EPT32)], kuf)
    pltpu.sync_copy(ki_hbm.at[pl.ds(base, EPT32)], kif)

    def fire(c, j):
        pltpu.async_copy(
            cu_hbm.at[kuf.at[pl.ds(c * CH, CH)]], gu4.at[j], sems[j])
        pltpu.async_copy(
            ci_hbm.at[kif.at[pl.ds(c * CH, CH)]], gi4.at[j], sems[j])

    def drain(j):
        pltpu.make_async_copy(
            cu_hbm.at[pl.ds(0, CH)], gu4.at[j], sems[j]).wait()
        pltpu.make_async_copy(
            ci_hbm.at[pl.ds(0, CH)], gi4.at[j], sems[j]).wait()

    def mult(c, j):
        for g in range(CH // L):
            cef[pl.ds(c * CH + g * L, L)] = (
                gu4[j, pl.ds(g * L, L)] * gi4[j, pl.ds(g * L, L)])

    for j in range(4):
        fire(j, j)

    def quad(q, _):
        for j in range(4):
            c = 4 * q + j
            drain(j)
            mult(c, j)

            @pl.when(c + 4 < 125)
            def _():
                fire(c + 4, j)
        return _

    lax.fori_loop(0, 31, quad, None)
    drain(0)
    mult(124, 0)
    pltpu.sync_copy(cef, ce_hbm.at[pl.ds(base, EPT32)])


def _bcast_lane(v16, j):
    idx = jnp.full((L, 1), j, jnp.int32)
    dnums = lax.GatherDimensionNumbers(
        offset_dims=(), collapsed_slice_dims=(0,), start_index_map=(0,))
    return lax.gather(v16, idx, dnums, (1,),
                      mode=lax.GatherScatterMode.PROMISE_IN_BOUNDS)


@functools.cache
def _msg_kernel():
  return functools.partial(
    pl.kernel,
    out_type=_sds((NU + NI, HID), jnp.float32),
    mesh=_mesh(),
    scratch_types=[
        pltpu.VMEM((EPT16,), jnp.int32),
        pltpu.VMEM((SUP,), jnp.int32),
        pltpu.VMEM((SUP,), jnp.float32),
        pltpu.VMEM((CH,), jnp.int32),
        pltpu.VMEM((CH, HID), jnp.float32),
        pltpu.VMEM((CH, HID), jnp.float32),
        pltpu.VMEM_SHARED((NU, HID), jnp.float32),
        pltpu.SemaphoreType.DMA,
        pltpu.SemaphoreType.DMA,
    ],
  )(_msg_body)


def _msg_body(hi_hbm, hu_hbm, ki_hbm, ku_hbm, ui_hbm, ii_hbm, ce_hbm, z_hbm,
              out_hbm, gidx, sidx, cef, s80, ga, gb, agg_s, gsa, gsb):
    cid = lax.axis_index("c")
    sid = lax.axis_index("s")
    base = sid * EPT16
    r0 = sid * 640
    nch = jnp.where(sid == NS - 1, 5, 8)
    npair = EPT16 // (2 * CH)
    npps = SUP // (2 * CH)

    def run(tab_hbm, g_hbm, s_hbm, out_base):
        pltpu.sync_copy(z_hbm, ga)

        def zc(k, _):
            pltpu.sync_copy(ga, agg_s.at[pl.ds(r0 + k * CH, CH)])
            return _

        lax.fori_loop(0, nch, zc, None)
        plsc.subcore_barrier()
        pltpu.sync_copy(g_hbm.at[pl.ds(base, EPT16)], gidx)

        def scale_scatter(buf, coff):
            def grp(g, _):
                cev = cef[pl.ds(coff + g * L, L)]
                s80[pl.ds(g * L, L)] = sidx[pl.ds(coff + g * L, L)]
                for j in range(L):
                    sc = _bcast_lane(cev, j)
                    r = g * L + j
                    for k in range(HID // L):
                        buf[r, pl.ds(k * L, L)] = buf[r, pl.ds(k * L, L)] * sc
                return _

            lax.fori_loop(0, CH // L, grp, None)
            pltpu.sync_copy(buf, agg_s.at[s80], add=True)

        pltpu.async_copy(tab_hbm.at[gidx.at[pl.ds(0, CH)]], ga, gsa)

        def pair(p, _):
            e0 = 2 * p * CH
            off0 = lax.rem(e0, SUP)

            @pl.when(lax.rem(p, npps) == 0)
            def _():
                pltpu.sync_copy(s_hbm.at[pl.ds(base + e0, SUP)], sidx)
                pltpu.sync_copy(ce_hbm.at[pl.ds(base + e0, SUP)], cef)

            pltpu.make_async_copy(
                tab_hbm.at[gidx.at[pl.ds(e0, CH)]], ga, gsa).wait()
            pltpu.async_copy(
                tab_hbm.at[gidx.at[pl.ds(e0 + CH, CH)]], gb, gsb)
            scale_scatter(ga, off0)

            @pl.when(p < npair - 1)
            def _():
                pltpu.async_copy(
                    tab_hbm.at[gidx.at[pl.ds(e0 + 2 * CH, CH)]], ga, gsa)

            pltpu.make_async_copy(
                tab_hbm.at[gidx.at[pl.ds(e0 + CH, CH)]], gb, gsb).wait()
            scale_scatter(gb, off0 + CH)
            return _

        lax.fori_loop(0, npair, pair, None)
        plsc.subcore_barrier()

        def oc(k, _):
            pltpu.sync_copy(agg_s.at[pl.ds(r0 + k * CH, CH)], ga)
            pltpu.sync_copy(
                ga, out_hbm.at[pl.ds(out_base + r0 + k * CH, CH)])
            return _

        lax.fori_loop(0, nch, oc, None)
        plsc.subcore_barrier()

    @pl.when(cid == 0)
    def _():
        run(hi_hbm, ki_hbm, ui_hbm, 0)

    @pl.when(cid == 1)
    def _():
        run(hu_hbm, ku_hbm, ii_hbm, NU)


def _mm_call(x, wstack, bias, relu_flag):
    rtot = x.shape[0]
    d = wstack.shape[-1]
    br = 2000
    nb = rtot // br
    half = nb // 2

    def body(f_ref, x_ref, w_ref, b_ref, o_ref):
        xv = x_ref[...]
        xv = jnp.where(f_ref[0, 0] > 0.5, jnp.maximum(xv, 0.0), xv)
        acc = jnp.dot(xv, w_ref[0], preferred_element_type=jnp.float32,
                      precision=lax.Precision.HIGHEST)
        o_ref[...] = acc + b_ref[...]

    return pl.pallas_call(
        body,
        grid=(nb,),
        in_specs=[
            pl.BlockSpec((1, 1), lambda i: (0, 0)),
            pl.BlockSpec((br, HID), lambda i: (i, 0)),
            pl.BlockSpec((1, HID, d), lambda i: (i // half, 0, 0)),
            pl.BlockSpec((1, d), lambda i: (0, 0)),
        ],
        out_specs=pl.BlockSpec((br, d), lambda i: (i, 0)),
        out_shape=_sds((rtot, d), jnp.float32),
    )(relu_flag, x, wstack, bias)


def kernel(ufeats, ifeats, edge_index, edge_type, Wu, Wi, W_h, b_h):
    u_idx = edge_index[0].astype(jnp.int32)
    i_idx = edge_index[1].astype(jnp.int32)
    et = edge_type.astype(jnp.int32)

    x0 = jnp.concatenate([ufeats, ifeats], axis=0)
    wst = [jnp.stack([Wu[l].transpose(1, 0, 2).reshape(HID, NT * HID),
                      Wi[l].transpose(1, 0, 2).reshape(HID, NT * HID)])
           for l in range(2)]
    zb640 = jnp.zeros((1, NT * HID), jnp.float32)
    wout = jnp.stack([W_h, W_h])
    bout = b_h.reshape(1, OUTD)

    zflat = jnp.zeros((STRIPE,), jnp.float32)
    z2d = jnp.zeros((CH, HID), jnp.float32)

    ku, ki, degu, degi = _deg_kernel()(u_idx, i_idx, et, zflat)
    cu, ci = _coef_call(degu, degi)
    ce = _ce_kernel()(ku, ki, cu.reshape(TABP), ci.reshape(TABP))

    wsc = jnp.stack(wst)
    flags = jnp.array([0.0, 1.0]).reshape(2, 1, 1)

    def layer(x, per):
        w_l, f_l = per
        y = _mm_call(x, w_l, zb640, f_l)
        hu_tab = y[:NU].reshape(TAB, HID)
        hi_tab = y[NU:].reshape(TAB, HID)
        agg = _msg_kernel()(hi_tab, hu_tab, ki, ku, u_idx, i_idx, ce, z2d)
        return agg, None

    aggf, _ = lax.scan(layer, x0, (wsc, flags))
    outc = _mm_call(aggf, wout, bout, jnp.ones((1, 1), jnp.float32))
    return outc[:NU], outc[NU:]
